# baseline (speedup 1.0000x reference)
import functools

import jax
import jax.numpy as jnp
from jax import lax
from jax.experimental import pallas as pl
from jax.experimental.pallas import tpu as pltpu
from jax.experimental.pallas import tpu_sc as plsc

_CONF = 0.5
_NMS = 0.4
_MAXD = 100
_NCLS = 80
_N = 20000
_NPAD = 20480
_CHUNK = 1280
_NSLICE = _CHUNK // 16


def _prep_body(p_ref, score_ref, x1_ref, y1_ref, x2_ref, y2_ref, cls_ref):
    cx = p_ref[0:1, :]
    cy = p_ref[1:2, :]
    w = p_ref[2:3, :]
    h = p_ref[3:4, :]
    obj = p_ref[4:5, :]
    cc = p_ref[5:85, :] * obj
    smax = jnp.max(cc, axis=0, keepdims=True)
    ids = jax.lax.broadcasted_iota(jnp.int32, (80, _CHUNK), 0)
    cls_i = jnp.min(jnp.where(cc == smax, ids, _NCLS), axis=0, keepdims=True)
    cls = cls_i.astype(jnp.float32)
    score_ref[...] = jnp.where(smax > _CONF, smax, 0.0)[None]
    off = cls * 4.0
    x1_ref[...] = ((cx - w / 2.0) + off)[None]
    y1_ref[...] = ((cy - h / 2.0) + off)[None]
    x2_ref[...] = ((cx + w / 2.0) + off)[None]
    y2_ref[...] = ((cy + h / 2.0) + off)[None]
    cls_ref[...] = cls[None]


def _perm16(v, idx):
    return v.at[idx].get(mode="promise_in_bounds")


def _allmax16(v, lane):
    for s in (1, 2, 4, 8):
        v = jnp.maximum(v, _perm16(v, lane ^ s))
    return v[0]


def _allmin16(v, lane):
    for s in (1, 2, 4, 8):
        v = jnp.minimum(v, _perm16(v, lane ^ s))
    return v[0]


def _nms_sc_body(score_hbm, x1_hbm, y1_hbm, x2_hbm, y2_hbm, cls_hbm, out_hbm,
                 s_v, x1_v, y1_v, x2_v, y2_v, cls_v, pub_stage, pub_all,
                 dets_v, pub_shared):
    cid = lax.axis_index("c")
    sid = lax.axis_index("s")

    @pl.when(cid == 0)
    def _core0():
        base = sid * _CHUNK
        pltpu.sync_copy(score_hbm.at[pl.ds(base, _CHUNK)], s_v)
        pltpu.sync_copy(x1_hbm.at[pl.ds(base, _CHUNK)], x1_v)
        pltpu.sync_copy(y1_hbm.at[pl.ds(base, _CHUNK)], y1_v)
        pltpu.sync_copy(x2_hbm.at[pl.ds(base, _CHUNK)], x2_v)
        pltpu.sync_copy(y2_hbm.at[pl.ds(base, _CHUNK)], y2_v)
        pltpu.sync_copy(cls_hbm.at[pl.ds(base, _CHUNK)], cls_v)

        @pl.when(sid == 0)
        def _zero_dets():
            z16 = jnp.zeros((16,), jnp.float32)

            def zd(k, _):
                dets_v[pl.ds(k * 16, 16)] = z16
                return ()

            lax.fori_loop(0, _MAXD, zd, ())

        lane = lax.broadcasted_iota(jnp.int32, (16,), 0)

        def round_fn(it, _):
            def amx(k, carry):
                bv, bk = carry
                v = s_v[pl.ds(k * 16, 16)]
                upd = v > bv
                return jnp.where(upd, v, bv), jnp.where(upd, k, bk)

            bv, bk = lax.fori_loop(
                0, _NSLICE, amx,
                (jnp.full((16,), -1.0, jnp.float32),
                 jnp.zeros((16,), jnp.int32)))
            lmax = _allmax16(bv, lane)
            cand = jnp.where(bv == lmax, bk * 16 + lane, _NPAD)
            lidx = _allmin16(cand, lane)
            wsl = pl.ds((lidx // 16) * 16, 16)
            wl = jnp.full((16,), lax.rem(lidx, 16), jnp.int32)
            rec = jnp.where(lane == 0, lmax, 0.0)
            rec = jnp.where(lane == 1, (base + lidx).astype(jnp.float32), rec)
            rec = jnp.where(lane == 2, _perm16(x1_v[wsl], wl), rec)
            rec = jnp.where(lane == 3, _perm16(y1_v[wsl], wl), rec)
            rec = jnp.where(lane == 4, _perm16(x2_v[wsl], wl), rec)
            rec = jnp.where(lane == 5, _perm16(y2_v[wsl], wl), rec)
            rec = jnp.where(lane == 6, _perm16(cls_v[wsl], wl), rec)
            pub_stage[...] = rec
            par = lax.rem(it, 2)
            pltpu.sync_copy(pub_stage, pub_shared.at[par, pl.ds(sid * 16, 16)])
            plsc.subcore_barrier()
            pltpu.sync_copy(pub_shared.at[par], pub_all)

            def red(t, carry):
                gv, gt = carry
                v = pub_all[pl.ds(t * 16, 16)][0]
                upd = v > gv
                return jnp.where(upd, v, gv), jnp.where(upd, t, gt)

            gmax, gtile = lax.fori_loop(0, 16, red,
                                        (jnp.float32(-1.0), jnp.int32(0)))
            wrec = pub_all[pl.ds(gtile * 16, 16)]
            widx = wrec[1].astype(jnp.int32)
            wx1 = wrec[2]
            wy1 = wrec[3]
            wx2 = wrec[4]
            wy2 = wrec[5]
            wcls = wrec[6]
            warea = (wx2 - wx1) * (wy2 - wy1)

            def supp(k, _):
                sl = pl.ds(k * 16, 16)
                x1s = x1_v[sl]
                y1s = y1_v[sl]
                x2s = x2_v[sl]
                y2s = y2_v[sl]
                xx1 = jnp.maximum(wx1, x1s)
                yy1 = jnp.maximum(wy1, y1s)
                xx2 = jnp.minimum(wx2, x2s)
                yy2 = jnp.minimum(wy2, y2s)
                zero = jnp.float32(0.0)
                inter = (jnp.maximum(xx2 - xx1, zero)
                         * jnp.maximum(yy2 - yy1, zero))
                areas = (x2s - x1s) * (y2s - y1s)
                iou = inter / (warea + areas - inter + 1e-9)
                sv = s_v[sl]
                s_v[sl] = jnp.where(iou > _NMS, zero, sv)
                return ()

            lax.fori_loop(0, _NSLICE, supp, ())

            @pl.when((widx >= base) & (widx < base + _CHUNK))
            def _self_supp():
                li = widx - base
                ks = (li // 16) * 16
                sl = pl.ds(ks, 16)
                s_v[sl] = jnp.where(lane == li - ks, 0.0, s_v[sl])

            @pl.when(sid == 0)
            def _record():
                valid = jnp.where(gmax > 0.0, jnp.float32(1.0),
                                  jnp.float32(0.0))
                woff = wcls * 4.0
                drow = jnp.where(lane == 0, (wx1 - woff) * valid, 0.0)
                drow = jnp.where(lane == 1, (wy1 - woff) * valid, drow)
                drow = jnp.where(lane == 2, (wx2 - woff) * valid, drow)
                drow = jnp.where(lane == 3, (wy2 - woff) * valid, drow)
                drow = jnp.where(lane == 4, gmax * valid, drow)
                drow = jnp.where(lane == 5, wcls * valid, drow)
                dets_v[pl.ds(it * 16, 16)] = drow

            return ()

        lax.fori_loop(0, _MAXD, round_fn, ())

        @pl.when(sid == 0)
        def _writeout():
            pltpu.sync_copy(dets_v, out_hbm)


def kernel(v3_out):
    pred_t = jnp.transpose(v3_out[0])
    pred_t = jnp.pad(pred_t, ((0, 3), (0, _NPAD - _N)))
    f32 = jnp.float32
    prep = pl.pallas_call(
        _prep_body,
        grid=(16,),
        in_specs=[pl.BlockSpec((88, _CHUNK), lambda i: (0, i))],
        out_specs=[pl.BlockSpec((1, 1, _CHUNK), lambda i: (i, 0, 0))] * 6,
        out_shape=[jax.ShapeDtypeStruct((16, 1, _CHUNK), f32)] * 6,
    )
    score, x1, y1, x2, y2, cls = (a.reshape(_NPAD) for a in prep(pred_t))

    nms = functools.partial(
        pl.kernel,
        mesh=plsc.VectorSubcoreMesh(core_axis_name="c", subcore_axis_name="s"),
        out_type=jax.ShapeDtypeStruct((_MAXD * 16,), f32),
        scratch_types=[
            pltpu.VMEM((_CHUNK,), f32),
            pltpu.VMEM((_CHUNK,), f32),
            pltpu.VMEM((_CHUNK,), f32),
            pltpu.VMEM((_CHUNK,), f32),
            pltpu.VMEM((_CHUNK,), f32),
            pltpu.VMEM((_CHUNK,), f32),
            pltpu.VMEM((16,), f32),
            pltpu.VMEM((256,), f32),
            pltpu.VMEM((_MAXD * 16,), f32),
            pltpu.VMEM_SHARED((2, 256), f32),
        ],
    )(_nms_sc_body)
    dets = nms(score, x1, y1, x2, y2, cls)
    return jax.lax.stop_gradient(dets.reshape(_MAXD, 16)[:, :6][None])


# speedup vs baseline: 1.1695x; 1.1695x over previous
import functools

import jax
import jax.numpy as jnp
from jax import lax
from jax.experimental import pallas as pl
from jax.experimental.pallas import tpu as pltpu
from jax.experimental.pallas import tpu_sc as plsc

_CONF = 0.5
_NMS = 0.4
_MAXD = 100
_NCLS = 80
_N = 20000
_NPAD = 20480
_CHUNK = 1280
_NSLICE = _CHUNK // 16


def _prep_body(p_ref, score_ref, x1_ref, y1_ref, x2_ref, y2_ref, cls_ref):
    p = jnp.transpose(p_ref[...])
    i = pl.program_id(0)
    gcol = (jax.lax.broadcasted_iota(jnp.int32, (1, _CHUNK), 1)
            + i * _CHUNK)
    live = gcol < _N
    cx = p[0:1, :]
    cy = p[1:2, :]
    w = p[2:3, :]
    h = p[3:4, :]
    obj = p[4:5, :]
    cc = p[5:85, :] * obj
    smax = jnp.max(cc, axis=0, keepdims=True)
    ids = jax.lax.broadcasted_iota(jnp.int32, (80, _CHUNK), 0)
    cls_i = jnp.min(jnp.where(cc == smax, ids, _NCLS), axis=0, keepdims=True)
    cls = jnp.where(live, cls_i.astype(jnp.float32), 0.0)
    score_ref[...] = jnp.where(live & (smax > _CONF), smax, 0.0)[None]
    off = cls * 4.0
    x1_ref[...] = jnp.where(live, (cx - w / 2.0) + off, 0.0)[None]
    y1_ref[...] = jnp.where(live, (cy - h / 2.0) + off, 0.0)[None]
    x2_ref[...] = jnp.where(live, (cx + w / 2.0) + off, 0.0)[None]
    y2_ref[...] = jnp.where(live, (cy + h / 2.0) + off, 0.0)[None]
    cls_ref[...] = cls[None]


def _perm16(v, idx):
    return v.at[idx].get(mode="promise_in_bounds")


def _allmax16(v, lane):
    for s in (1, 2, 4, 8):
        v = jnp.maximum(v, _perm16(v, lane ^ s))
    return v[0]


def _allmin16(v, lane):
    for s in (1, 2, 4, 8):
        v = jnp.minimum(v, _perm16(v, lane ^ s))
    return v[0]


def _nms_sc_body(score_hbm, x1_hbm, y1_hbm, x2_hbm, y2_hbm, cls_hbm, out_hbm,
                 s_v, x1_v, y1_v, x2_v, y2_v, cls_v, pub_stage, pub_all,
                 dets_v, pub_shared):
    cid = lax.axis_index("c")
    sid = lax.axis_index("s")

    @pl.when(cid == 0)
    def _core0():
        base = sid * _CHUNK
        pltpu.sync_copy(score_hbm.at[pl.ds(base, _CHUNK)], s_v)
        pltpu.sync_copy(x1_hbm.at[pl.ds(base, _CHUNK)], x1_v)
        pltpu.sync_copy(y1_hbm.at[pl.ds(base, _CHUNK)], y1_v)
        pltpu.sync_copy(x2_hbm.at[pl.ds(base, _CHUNK)], x2_v)
        pltpu.sync_copy(y2_hbm.at[pl.ds(base, _CHUNK)], y2_v)
        pltpu.sync_copy(cls_hbm.at[pl.ds(base, _CHUNK)], cls_v)

        @pl.when(sid == 0)
        def _zero_dets():
            z16 = jnp.zeros((16,), jnp.float32)

            def zd(k, _):
                dets_v[pl.ds(k * 16, 16)] = z16
                return ()

            lax.fori_loop(0, _MAXD, zd, ())

        lane = lax.broadcasted_iota(jnp.int32, (16,), 0)

        bv0 = jnp.full((16,), -1.0, jnp.float32)
        bk0 = jnp.zeros((16,), jnp.int32)
        for k in range(_NSLICE):
            v = s_v[pl.ds(k * 16, 16)]
            upd = v > bv0
            bv0 = jnp.where(upd, v, bv0)
            bk0 = jnp.where(upd, k, bk0)

        def round_fn(it, carry):
            bv, bk = carry
            lmax = _allmax16(bv, lane)
            cand = jnp.where(bv == lmax, bk * 16 + lane, _NPAD)
            lidx = _allmin16(cand, lane)
            wsl = pl.ds((lidx // 16) * 16, 16)
            wl = jnp.full((16,), lax.rem(lidx, 16), jnp.int32)
            rec = jnp.where(lane == 0, lmax, 0.0)
            rec = jnp.where(lane == 1, (base + lidx).astype(jnp.float32), rec)
            rec = jnp.where(lane == 2, _perm16(x1_v[wsl], wl), rec)
            rec = jnp.where(lane == 3, _perm16(y1_v[wsl], wl), rec)
            rec = jnp.where(lane == 4, _perm16(x2_v[wsl], wl), rec)
            rec = jnp.where(lane == 5, _perm16(y2_v[wsl], wl), rec)
            rec = jnp.where(lane == 6, _perm16(cls_v[wsl], wl), rec)
            pub_stage[...] = rec
            par = lax.rem(it, 2)
            pltpu.sync_copy(pub_stage, pub_shared.at[par, pl.ds(sid * 16, 16)])
            plsc.subcore_barrier()
            pltpu.sync_copy(pub_shared.at[par], pub_all)

            gmax = pub_all[pl.ds(0, 16)][0]
            gtile = jnp.int32(0)
            for t in range(1, 16):
                v = pub_all[pl.ds(t * 16, 16)][0]
                upd = v > gmax
                gmax = jnp.where(upd, v, gmax)
                gtile = jnp.where(upd, jnp.int32(t), gtile)
            wrec = pub_all[pl.ds(gtile * 16, 16)]
            widx = wrec[1].astype(jnp.int32)
            wx1 = wrec[2]
            wy1 = wrec[3]
            wx2 = wrec[4]
            wy2 = wrec[5]
            wcls = wrec[6]
            warea = (wx2 - wx1) * (wy2 - wy1)

            zero = jnp.float32(0.0)
            nbv = jnp.full((16,), -1.0, jnp.float32)
            nbk = jnp.zeros((16,), jnp.int32)
            lwidx = widx - base
            for k in range(_NSLICE):
                sl = pl.ds(k * 16, 16)
                x1s = x1_v[sl]
                y1s = y1_v[sl]
                x2s = x2_v[sl]
                y2s = y2_v[sl]
                xx1 = jnp.maximum(wx1, x1s)
                yy1 = jnp.maximum(wy1, y1s)
                xx2 = jnp.minimum(wx2, x2s)
                yy2 = jnp.minimum(wy2, y2s)
                inter = (jnp.maximum(xx2 - xx1, zero)
                         * jnp.maximum(yy2 - yy1, zero))
                areas = (x2s - x1s) * (y2s - y1s)
                iou = inter / (warea + areas - inter + 1e-9)
                sv = s_v[sl]
                kill = (iou > _NMS) | (lane + (k * 16) == lwidx)
                nsv = jnp.where(kill, zero, sv)
                s_v[sl] = nsv
                upd = nsv > nbv
                nbv = jnp.where(upd, nsv, nbv)
                nbk = jnp.where(upd, k, nbk)

            @pl.when(sid == 0)
            def _record():
                valid = jnp.where(gmax > 0.0, jnp.float32(1.0),
                                  jnp.float32(0.0))
                woff = wcls * 4.0
                drow = jnp.where(lane == 0, (wx1 - woff) * valid, 0.0)
                drow = jnp.where(lane == 1, (wy1 - woff) * valid, drow)
                drow = jnp.where(lane == 2, (wx2 - woff) * valid, drow)
                drow = jnp.where(lane == 3, (wy2 - woff) * valid, drow)
                drow = jnp.where(lane == 4, gmax * valid, drow)
                drow = jnp.where(lane == 5, wcls * valid, drow)
                dets_v[pl.ds(it * 16, 16)] = drow

            return nbv, nbk

        lax.fori_loop(0, _MAXD, round_fn, (bv0, bk0))

        @pl.when(sid == 0)
        def _writeout():
            pltpu.sync_copy(dets_v, out_hbm)


def kernel(v3_out):
    pred = v3_out[0]
    f32 = jnp.float32
    prep = pl.pallas_call(
        _prep_body,
        grid=(16,),
        in_specs=[pl.BlockSpec((_CHUNK, 85), lambda i: (i, 0))],
        out_specs=[pl.BlockSpec((1, 1, _CHUNK), lambda i: (i, 0, 0))] * 6,
        out_shape=[jax.ShapeDtypeStruct((16, 1, _CHUNK), f32)] * 6,
    )
    score, x1, y1, x2, y2, cls = (a.reshape(_NPAD) for a in prep(pred))

    nms = functools.partial(
        pl.kernel,
        mesh=plsc.VectorSubcoreMesh(core_axis_name="c", subcore_axis_name="s"),
        out_type=jax.ShapeDtypeStruct((_MAXD * 16,), f32),
        scratch_types=[
            pltpu.VMEM((_CHUNK,), f32),
            pltpu.VMEM((_CHUNK,), f32),
            pltpu.VMEM((_CHUNK,), f32),
            pltpu.VMEM((_CHUNK,), f32),
            pltpu.VMEM((_CHUNK,), f32),
            pltpu.VMEM((_CHUNK,), f32),
            pltpu.VMEM((16,), f32),
            pltpu.VMEM((256,), f32),
            pltpu.VMEM((_MAXD * 16,), f32),
            pltpu.VMEM_SHARED((2, 256), f32),
        ],
    )(_nms_sc_body)
    dets = nms(score, x1, y1, x2, y2, cls)
    return jax.lax.stop_gradient(dets.reshape(_MAXD, 16)[:, :6][None])
